# Initial kernel scaffold; baseline (speedup 1.0000x reference)
import jax
import jax.numpy as jnp
from jax.experimental import pallas as pl


def kernel(world_pos, prev_world_pos, node_type, cells, mesh_pos, is_training, W1, b1, W2, b2):
    raise NotImplementedError("write your pallas kernel here")


# speedup vs baseline: 2.1620x; 2.1620x over previous
import functools

import jax
import jax.numpy as jnp
from jax.experimental import pallas as pl

_NTYPE = 9


def _combine_body(dis_ref, g_ref, z_ref, out_ref):
    dis = dis_ref[...]
    out_ref[...] = dis * g_ref[...] + 2.0 * dis * z_ref[...]


def _combine(dis2d, g, z):
    n = g.shape[0]
    blk = 2000
    return pl.pallas_call(
        _combine_body,
        grid=(n // blk,),
        in_specs=[
            pl.BlockSpec((blk, 1), lambda i: (i, 0)),
            pl.BlockSpec((blk, g.shape[1]), lambda i: (i, 0)),
            pl.BlockSpec((blk, g.shape[1]), lambda i: (i, 0)),
        ],
        out_specs=pl.BlockSpec((blk, g.shape[1]), lambda i: (i, 0)),
        out_shape=jax.ShapeDtypeStruct(g.shape, g.dtype),
    )(dis2d, g, z)


def _layer(x, W, b, dis2d, s_d, r_d, w, n):
    y = x @ W.T + b
    z = dis2d * y
    g = jnp.zeros((n, y.shape[1]), y.dtype)
    g = g.at[s_d].add(w[:, None] * z[r_d], mode="drop")
    g = g.at[r_d].add(w[:, None] * z[s_d], mode="drop")
    return _combine(dis2d, g, z)


def kernel(world_pos, prev_world_pos, node_type, cells, mesh_pos, is_training, W1, b1, W2, b2):
    n = world_pos.shape[0]
    velocity = world_pos - prev_world_pos
    onehot = jax.nn.one_hot(node_type[:, 0], _NTYPE, dtype=world_pos.dtype)
    node_features = jnp.concatenate([velocity, onehot], axis=-1)

    e = jnp.concatenate(
        [cells[:, 0:2], cells[:, 1:3],
         jnp.stack([cells[:, 2], cells[:, 0]], axis=1)], axis=0)
    lo = jnp.min(e, axis=1).astype(jnp.uint32)
    hi = jnp.max(e, axis=1).astype(jnp.uint32)
    key = (hi << 16) | lo
    sk = jnp.sort(key)
    first = jnp.concatenate(
        [jnp.ones((1,), jnp.bool_), sk[1:] != sk[:-1]])
    w = first.astype(jnp.float32)
    s_d = (sk >> 16).astype(jnp.int32)
    r_d = (sk & jnp.uint32(0xFFFF)).astype(jnp.int32)

    deg = jnp.full((n,), 2.0, jnp.float32)
    deg = deg.at[s_d].add(w, mode="drop")
    deg = deg.at[r_d].add(w, mode="drop")
    dis2d = jax.lax.rsqrt(deg)[:, None]

    h = _layer(node_features, W1, b1, dis2d, s_d, r_d, w, n)
    h = _layer(h, W2, b2, dis2d, s_d, r_d, w, n)
    return jnp.where(is_training != 0, h, 2.0 * world_pos + h - prev_world_pos)


# speedup vs baseline: 18.2731x; 8.4519x over previous
import functools

import jax
import jax.numpy as jnp
from jax import lax
from jax.experimental import pallas as pl
from jax.experimental.pallas import tpu as pltpu
from jax.experimental.pallas import tpu_sc as plsc

_NTYPE = 9
_N = 50000
_NPAD = 50176
_E = 300000
_EPAD = 307200
_NC = 2
_NS = 16
_ECH = _EPAD // (_NC * _NS)
_PT = _NPAD // _NS

_MESH = plsc.VectorSubcoreMesh(
    core_axis_name="c", subcore_axis_name="s", num_cores=_NC, num_subcores=_NS)


def _zero_fill(buf, nwords):
    @pl.loop(0, nwords // 16)
    def _(i):
        buf[pl.ds(i * 16, 16)] = jnp.zeros((16,), buf.dtype)


def _deg_body(sidx_hbm, ridx_hbm, w_hbm, deg_hbm, sidx_v, ridx_v, w_v, zbuf, deg_sp):
    cid = lax.axis_index("c")
    sid = lax.axis_index("s")
    base = (sid * _NC + cid) * _ECH
    pltpu.sync_copy(sidx_hbm.at[pl.ds(base, _ECH)], sidx_v)
    pltpu.sync_copy(ridx_hbm.at[pl.ds(base, _ECH)], ridx_v)
    pltpu.sync_copy(w_hbm.at[pl.ds(base, _ECH)], w_v)
    _zero_fill(zbuf, _PT)
    pltpu.sync_copy(zbuf, deg_sp.at[pl.ds(sid * _PT, _PT)])
    plsc.subcore_barrier()
    pltpu.sync_copy(w_v, deg_sp.at[sidx_v], add=True)
    pltpu.sync_copy(w_v, deg_sp.at[ridx_v], add=True)
    plsc.subcore_barrier()
    pltpu.sync_copy(deg_sp.at[pl.ds(sid * _PT, _PT)], zbuf)
    pltpu.sync_copy(zbuf, deg_hbm.at[pl.ds(cid * _NPAD + sid * _PT, _PT)])


_deg_call = pl.kernel(
    _deg_body,
    out_type=jax.ShapeDtypeStruct((_NC * _NPAD,), jnp.float32),
    mesh=_MESH,
    scratch_types=[
        pltpu.VMEM((_ECH,), jnp.int32),
        pltpu.VMEM((_ECH,), jnp.int32),
        pltpu.VMEM((_ECH,), jnp.float32),
        pltpu.VMEM((_PT,), jnp.float32),
        pltpu.VMEM_SHARED((_NPAD,), jnp.float32),
    ],
)


def _msg_body(sidx_hbm, ridx_hbm, w_hbm, z0_hbm, z1_hbm, z2_hbm, g_hbm,
              sidx_v, ridx_v, w_v, zg_v, vs_v, zbuf, g0, g1, g2):
    cid = lax.axis_index("c")
    sid = lax.axis_index("s")
    base = (sid * _NC + cid) * _ECH
    pltpu.sync_copy(sidx_hbm.at[pl.ds(base, _ECH)], sidx_v)
    pltpu.sync_copy(ridx_hbm.at[pl.ds(base, _ECH)], ridx_v)
    pltpu.sync_copy(w_hbm.at[pl.ds(base, _ECH)], w_v)
    _zero_fill(zbuf, _PT)
    gtabs = (g0, g1, g2)
    for gtab in gtabs:
        pltpu.sync_copy(zbuf, gtab.at[pl.ds(sid * _PT, _PT)])
    plsc.subcore_barrier()
    for z_hbm, gtab in zip((z0_hbm, z1_hbm, z2_hbm), gtabs):
        for src_v, dst_v in ((ridx_v, sidx_v), (sidx_v, ridx_v)):
            pltpu.sync_copy(z_hbm.at[src_v], zg_v)

            @pl.loop(0, _ECH // 16)
            def _(i):
                vs_v[pl.ds(i * 16, 16)] = (
                    w_v[pl.ds(i * 16, 16)] * zg_v[pl.ds(i * 16, 16)])

            pltpu.sync_copy(vs_v, gtab.at[dst_v], add=True)
    plsc.subcore_barrier()
    for c, gtab in enumerate(gtabs):
        pltpu.sync_copy(gtab.at[pl.ds(sid * _PT, _PT)], zbuf)
        pltpu.sync_copy(zbuf, g_hbm.at[pl.ds((cid * 3 + c) * _NPAD + sid * _PT, _PT)])


_msg_call = pl.kernel(
    _msg_body,
    out_type=jax.ShapeDtypeStruct((_NC * 3 * _NPAD,), jnp.float32),
    mesh=_MESH,
    scratch_types=[
        pltpu.VMEM((_ECH,), jnp.int32),
        pltpu.VMEM((_ECH,), jnp.int32),
        pltpu.VMEM((_ECH,), jnp.float32),
        pltpu.VMEM((_ECH,), jnp.float32),
        pltpu.VMEM((_ECH,), jnp.float32),
        pltpu.VMEM((_PT,), jnp.float32),
        pltpu.VMEM_SHARED((_NPAD,), jnp.float32),
        pltpu.VMEM_SHARED((_NPAD,), jnp.float32),
        pltpu.VMEM_SHARED((_NPAD,), jnp.float32),
    ],
)


def _prep_body(wpT_ref, pwpT_ref, ntT_ref, deg_ref, W1_ref, b1_ref, z_ref, dis_ref):
    vel = wpT_ref[...] - pwpT_ref[...]
    rows = lax.broadcasted_iota(jnp.int32, (_NTYPE, _NPAD), 0)
    onehot = (rows == jnp.broadcast_to(ntT_ref[...], (_NTYPE, _NPAD))).astype(jnp.float32)
    nfT = jnp.concatenate([vel, onehot], axis=0)
    y = lax.dot_general(W1_ref[...], nfT, (((1,), (0,)), ((), ())),
                        preferred_element_type=jnp.float32)
    deg = deg_ref[0:1, :] + deg_ref[1:2, :] + 2.0
    dis = lax.rsqrt(deg)
    z_ref[...] = dis * (y + b1_ref[...])
    dis_ref[...] = dis


def _prep_call(wpT, pwpT, ntT, deg2, W1, b1col):
    return pl.pallas_call(
        _prep_body,
        out_shape=[jax.ShapeDtypeStruct((3, _NPAD), jnp.float32),
                   jax.ShapeDtypeStruct((1, _NPAD), jnp.float32)],
    )(wpT, pwpT, ntT, deg2, W1, b1col)


def _mid_body(g_ref, dis_ref, z_ref, W2_ref, b2_ref, z2_ref):
    dis = dis_ref[...]
    gt = g_ref[0:3, :] + g_ref[3:6, :]
    h = dis * gt + 2.0 * dis * z_ref[...]
    y2 = lax.dot_general(W2_ref[...], h, (((1,), (0,)), ((), ())),
                         preferred_element_type=jnp.float32)
    z2_ref[...] = dis * (y2 + b2_ref[...])


def _mid_call(g6, dis, z, W2, b2col):
    return pl.pallas_call(
        _mid_body,
        out_shape=jax.ShapeDtypeStruct((3, _NPAD), jnp.float32),
    )(g6, dis, z, W2, b2col)


def _fin_body(g_ref, dis_ref, z_ref, out_ref):
    dis = dis_ref[...]
    gt = g_ref[0:3, :] + g_ref[3:6, :]
    out_ref[...] = dis * gt + 2.0 * dis * z_ref[...]


def _fin_call(g6, dis, z):
    return pl.pallas_call(
        _fin_body,
        out_shape=jax.ShapeDtypeStruct((3, _NPAD), jnp.float32),
    )(g6, dis, z)


def kernel(world_pos, prev_world_pos, node_type, cells, mesh_pos, is_training, W1, b1, W2, b2):
    e = jnp.concatenate(
        [cells[:, 0:2], cells[:, 1:3],
         jnp.stack([cells[:, 2], cells[:, 0]], axis=1)], axis=0)
    lo = jnp.min(e, axis=1).astype(jnp.uint32)
    hi = jnp.max(e, axis=1).astype(jnp.uint32)
    key = (hi << 16) | lo
    keyp = jnp.concatenate([key, jnp.broadcast_to(key[0], (_EPAD - _E,))])
    sk = jnp.sort(keyp)
    prev = jnp.concatenate([sk[:1] ^ jnp.uint32(1), sk[:-1]])
    w = (sk != prev).astype(jnp.float32)
    sidx = (sk >> 16).astype(jnp.int32)
    ridx = (sk & jnp.uint32(0xFFFF)).astype(jnp.int32)

    pad = ((0, _NPAD - _N), (0, 0))
    wpT = jnp.pad(world_pos, pad).T
    pwpT = jnp.pad(prev_world_pos, pad).T
    ntT = jnp.pad(node_type, pad).T
    b1col = b1[:, None]
    b2col = b2[:, None]

    deg2 = _deg_call(sidx, ridx, w).reshape(_NC, _NPAD)
    z1, dis = _prep_call(wpT, pwpT, ntT, deg2, W1, b1col)

    g1 = _msg_call(sidx, ridx, w, z1[0], z1[1], z1[2]).reshape(_NC * 3, _NPAD)
    z2 = _mid_call(g1, dis, z1, W2, b2col)

    g2 = _msg_call(sidx, ridx, w, z2[0], z2[1], z2[2]).reshape(_NC * 3, _NPAD)
    h2T = _fin_call(g2, dis, z2)

    h = h2T[:, :_N].T
    return jnp.where(is_training != 0, h, 2.0 * world_pos + h - prev_world_pos)


# speedup vs baseline: 25.5279x; 1.3970x over previous
import functools

import jax
import jax.numpy as jnp
from jax import lax
from jax.experimental import pallas as pl
from jax.experimental.pallas import tpu as pltpu
from jax.experimental.pallas import tpu_sc as plsc

_NTYPE = 9
_N = 50000
_NPAD = 50176
_E = 300000
_EPAD = 307200
_NC = 2
_NS = 16
_ECH = _EPAD // (_NC * _NS)
_PT = _NPAD // _NS

_MESH = plsc.VectorSubcoreMesh(
    core_axis_name="c", subcore_axis_name="s", num_cores=_NC, num_subcores=_NS)


def _zero_fill(buf, nwords):
    @pl.loop(0, nwords // 16)
    def _(i):
        buf[pl.ds(i * 16, 16)] = jnp.zeros((16,), buf.dtype)


def _deg_body(sidx_hbm, ridx_hbm, w_hbm, deg_hbm, sidx_v, ridx_v, w_v, zbuf, deg_sp):
    cid = lax.axis_index("c")
    sid = lax.axis_index("s")
    base = (sid * _NC + cid) * _ECH
    pltpu.sync_copy(sidx_hbm.at[pl.ds(base, _ECH)], sidx_v)
    pltpu.sync_copy(ridx_hbm.at[pl.ds(base, _ECH)], ridx_v)
    pltpu.sync_copy(w_hbm.at[pl.ds(base, _ECH)], w_v)
    _zero_fill(zbuf, _PT)
    pltpu.sync_copy(zbuf, deg_sp.at[pl.ds(sid * _PT, _PT)])
    plsc.subcore_barrier()
    pltpu.sync_copy(w_v, deg_sp.at[sidx_v], add=True)
    pltpu.sync_copy(w_v, deg_sp.at[ridx_v], add=True)
    plsc.subcore_barrier()
    pltpu.sync_copy(deg_sp.at[pl.ds(sid * _PT, _PT)], zbuf)
    pltpu.sync_copy(zbuf, deg_hbm.at[pl.ds(cid * _NPAD + sid * _PT, _PT)])


_deg_call = pl.kernel(
    _deg_body,
    out_type=jax.ShapeDtypeStruct((_NC * _NPAD,), jnp.float32),
    mesh=_MESH,
    scratch_types=[
        pltpu.VMEM((_ECH,), jnp.int32),
        pltpu.VMEM((_ECH,), jnp.int32),
        pltpu.VMEM((_ECH,), jnp.float32),
        pltpu.VMEM((_PT,), jnp.float32),
        pltpu.VMEM_SHARED((_NPAD,), jnp.float32),
    ],
)


def _msg_body(sidx_hbm, ridx_hbm, w_hbm, z0_hbm, z1_hbm, z2_hbm, g_hbm,
              sidx_v, ridx_v, w_v, zg0, zg1, zg2, zg3, zg4, zg5, zbuf,
              g0, g1, g2, s0, s1, s2, s3, s4, s5):
    cid = lax.axis_index("c")
    sid = lax.axis_index("s")
    base = (sid * _NC + cid) * _ECH
    pltpu.sync_copy(sidx_hbm.at[pl.ds(base, _ECH)], sidx_v)
    pltpu.sync_copy(ridx_hbm.at[pl.ds(base, _ECH)], ridx_v)
    pltpu.sync_copy(w_hbm.at[pl.ds(base, _ECH)], w_v)
    zgs = (zg0, zg1, zg2, zg3, zg4, zg5)
    sems = (s0, s1, s2, s3, s4, s5)
    gtabs = (g0, g1, g2)
    gathers = []
    for c, z_hbm in enumerate((z0_hbm, z1_hbm, z2_hbm)):
        gathers.append(pltpu.async_copy(z_hbm.at[ridx_v], zgs[2 * c], sems[2 * c]))
        gathers.append(pltpu.async_copy(z_hbm.at[sidx_v], zgs[2 * c + 1], sems[2 * c + 1]))
    _zero_fill(zbuf, _PT)
    for gtab in gtabs:
        pltpu.sync_copy(zbuf, gtab.at[pl.ds(sid * _PT, _PT)])
    plsc.subcore_barrier()
    scatters = []
    for j in range(6):
        gathers[j].wait()
        zg = zgs[j]

        @pl.loop(0, _ECH // 16)
        def _(i):
            zg[pl.ds(i * 16, 16)] = (
                w_v[pl.ds(i * 16, 16)] * zg[pl.ds(i * 16, 16)])

        dst = sidx_v if j % 2 == 0 else ridx_v
        scatters.append(
            pltpu.async_copy(zg, gtabs[j // 2].at[dst], sems[j], add=True))
    for sc in scatters:
        sc.wait()
    plsc.subcore_barrier()
    for c, gtab in enumerate(gtabs):
        pltpu.sync_copy(gtab.at[pl.ds(sid * _PT, _PT)], zbuf)
        pltpu.sync_copy(zbuf, g_hbm.at[pl.ds((cid * 3 + c) * _NPAD + sid * _PT, _PT)])


_msg_call = pl.kernel(
    _msg_body,
    out_type=jax.ShapeDtypeStruct((_NC * 3 * _NPAD,), jnp.float32),
    mesh=_MESH,
    scratch_types=(
        [pltpu.VMEM((_ECH,), jnp.int32)] * 2
        + [pltpu.VMEM((_ECH,), jnp.float32)] * 7
        + [pltpu.VMEM((_PT,), jnp.float32)]
        + [pltpu.VMEM_SHARED((_NPAD,), jnp.float32)] * 3
        + [pltpu.SemaphoreType.DMA] * 6
    ),
)


def _prep_body(wpT_ref, pwpT_ref, ntT_ref, deg_ref, W1_ref, b1_ref, z_ref, dis_ref):
    vel = wpT_ref[...] - pwpT_ref[...]
    rows = lax.broadcasted_iota(jnp.int32, (_NTYPE, _NPAD), 0)
    onehot = (rows == jnp.broadcast_to(ntT_ref[...], (_NTYPE, _NPAD))).astype(jnp.float32)
    nfT = jnp.concatenate([vel, onehot], axis=0)
    y = lax.dot_general(W1_ref[...], nfT, (((1,), (0,)), ((), ())),
                        preferred_element_type=jnp.float32)
    deg = deg_ref[0:1, :] + deg_ref[1:2, :] + 2.0
    dis = lax.rsqrt(deg)
    z_ref[...] = dis * (y + b1_ref[...])
    dis_ref[...] = dis


def _prep_call(wpT, pwpT, ntT, deg2, W1, b1col):
    return pl.pallas_call(
        _prep_body,
        out_shape=[jax.ShapeDtypeStruct((3, _NPAD), jnp.float32),
                   jax.ShapeDtypeStruct((1, _NPAD), jnp.float32)],
    )(wpT, pwpT, ntT, deg2, W1, b1col)


def _mid_body(g_ref, dis_ref, z_ref, W2_ref, b2_ref, z2_ref):
    dis = dis_ref[...]
    gt = g_ref[0:3, :] + g_ref[3:6, :]
    h = dis * gt + 2.0 * dis * z_ref[...]
    y2 = lax.dot_general(W2_ref[...], h, (((1,), (0,)), ((), ())),
                         preferred_element_type=jnp.float32)
    z2_ref[...] = dis * (y2 + b2_ref[...])


def _mid_call(g6, dis, z, W2, b2col):
    return pl.pallas_call(
        _mid_body,
        out_shape=jax.ShapeDtypeStruct((3, _NPAD), jnp.float32),
    )(g6, dis, z, W2, b2col)


def _fin_body(g_ref, dis_ref, z_ref, out_ref):
    dis = dis_ref[...]
    gt = g_ref[0:3, :] + g_ref[3:6, :]
    out_ref[...] = dis * gt + 2.0 * dis * z_ref[...]


def _fin_call(g6, dis, z):
    return pl.pallas_call(
        _fin_body,
        out_shape=jax.ShapeDtypeStruct((3, _NPAD), jnp.float32),
    )(g6, dis, z)


def kernel(world_pos, prev_world_pos, node_type, cells, mesh_pos, is_training, W1, b1, W2, b2):
    e = jnp.concatenate(
        [cells[:, 0:2], cells[:, 1:3],
         jnp.stack([cells[:, 2], cells[:, 0]], axis=1)], axis=0)
    lo = jnp.min(e, axis=1).astype(jnp.uint32)
    hi = jnp.max(e, axis=1).astype(jnp.uint32)
    key = (hi << 16) | lo
    keyp = jnp.concatenate([key, jnp.broadcast_to(key[0], (_EPAD - _E,))])
    sk = jnp.sort(keyp)
    prev = jnp.concatenate([sk[:1] ^ jnp.uint32(1), sk[:-1]])
    w = (sk != prev).astype(jnp.float32)
    sidx = (sk >> 16).astype(jnp.int32)
    ridx = (sk & jnp.uint32(0xFFFF)).astype(jnp.int32)

    pad = ((0, _NPAD - _N), (0, 0))
    wpT = jnp.pad(world_pos, pad).T
    pwpT = jnp.pad(prev_world_pos, pad).T
    ntT = jnp.pad(node_type, pad).T
    b1col = b1[:, None]
    b2col = b2[:, None]

    deg2 = _deg_call(sidx, ridx, w).reshape(_NC, _NPAD)
    z1, dis = _prep_call(wpT, pwpT, ntT, deg2, W1, b1col)

    g1 = _msg_call(sidx, ridx, w, z1[0], z1[1], z1[2]).reshape(_NC * 3, _NPAD)
    z2 = _mid_call(g1, dis, z1, W2, b2col)

    g2 = _msg_call(sidx, ridx, w, z2[0], z2[1], z2[2]).reshape(_NC * 3, _NPAD)
    h2T = _fin_call(g2, dis, z2)

    h = h2T[:, :_N].T
    return jnp.where(is_training != 0, h, 2.0 * world_pos + h - prev_world_pos)
